# baseline (speedup 1.0000x reference)
import functools

import jax
import jax.numpy as jnp
from jax import lax
from jax.experimental import pallas as pl
from jax.experimental.pallas import tpu as pltpu
from jax.experimental.pallas import tpu_sc as plsc

N = 10000
NPAD = 10240
E = 160000
EPAD = 163840
G = 8
F = 128
CH = 128
NC, NS = 2, 16
RPT = NPAD // NS

_mesh = lambda: plsc.VectorSubcoreMesh(
    core_axis_name="c", subcore_axis_name="s", num_cores=NC, num_subcores=NS)


CHW = 64
NCHW = EPAD // NS // CHW


def _make_wide_agg(npass):
  nf = NC * npass

  def body(table, srcoff, dstf, out, gidx, dv0, dv1, rows0, rows1, acc,
           gsem0, gsem1, ssem0, ssem1):
    c = lax.axis_index("c")
    s = lax.axis_index("s")
    row0 = s * RPT
    for p in range(npass):
      f = c * npass + p
      pltpu.sync_copy(srcoff.at[f, s], gidx)
      pltpu.sync_copy(table.at[pl.ds(f * NPAD + row0, RPT)],
                      acc.at[pl.ds(row0, RPT)])
      plsc.subcore_barrier()

      pltpu.async_copy(table.at[gidx.at[0]], rows0, gsem0)

      def chunk2(i, carry):
        j0 = 2 * i
        base = s * (NCHW * CHW)
        pltpu.sync_copy(dstf.at[pl.ds(base + j0 * CHW, CHW)], dv0)
        pltpu.sync_copy(dstf.at[pl.ds(base + (j0 + 1) * CHW, CHW)], dv1)
        pltpu.make_async_copy(table.at[gidx.at[j0]], rows0, gsem0).wait()
        pltpu.async_copy(table.at[gidx.at[j0 + 1]], rows1, gsem1)
        sd0 = pltpu.async_copy(rows0, acc.at[dv0], ssem0, add=True)
        pltpu.make_async_copy(table.at[gidx.at[j0 + 1]], rows1, gsem1).wait()
        sd0.wait()
        pltpu.async_copy(table.at[gidx.at[j0 + 2]], rows0, gsem0)
        sd1 = pltpu.async_copy(rows1, acc.at[dv1], ssem1, add=True)
        sd1.wait()
        return carry

      lax.fori_loop(0, NCHW // 2, chunk2, 0)
      pltpu.make_async_copy(table.at[gidx.at[NCHW]], rows0, gsem0).wait()
      plsc.subcore_barrier()
      pltpu.sync_copy(acc.at[pl.ds(row0, RPT)], out.at[pl.ds(f * NPAD + row0, RPT)])
      plsc.subcore_barrier()

  return pl.kernel(
      body,
      out_type=jax.ShapeDtypeStruct((nf * NPAD, F), jnp.float32),
      mesh=_mesh(),
      scratch_types=[
          pltpu.VMEM((NCHW + 1, CHW), jnp.int32),
          pltpu.VMEM((CHW,), jnp.int32),
          pltpu.VMEM((CHW,), jnp.int32),
          pltpu.VMEM((CHW, F), jnp.float32),
          pltpu.VMEM((CHW, F), jnp.float32),
          pltpu.VMEM_SHARED((NPAD, F), jnp.float32),
          pltpu.SemaphoreType.DMA,
          pltpu.SemaphoreType.DMA,
          pltpu.SemaphoreType.DMA,
          pltpu.SemaphoreType.DMA,
      ],
  )


def _make_scalar_agg(gather, width=F):
  ept = EPAD // (NC * NS)
  nch = ept // CH

  def body(table, srcp, dstp, out, src_v, dst_v, rows_v, acc, gsem, ssem):
    c = lax.axis_index("c")
    s = lax.axis_index("s")
    row0 = s * RPT
    wid = s * NC + c
    pltpu.sync_copy(table.at[pl.ds(row0, RPT)], acc.at[pl.ds(row0, RPT)])
    if not gather:
      pltpu.sync_copy(table.at[pl.ds(0, CH)], rows_v)
    plsc.subcore_barrier()

    def chunk(i, carry):
      off = wid * ept + i * CH
      pltpu.sync_copy(srcp.at[pl.ds(off, CH)], src_v)
      pltpu.sync_copy(dstp.at[pl.ds(off, CH)], dst_v)
      if gather:
        pltpu.async_copy(table.at[src_v], rows_v, gsem).wait()
      pltpu.async_copy(rows_v, acc.at[dst_v], ssem, add=True).wait()
      return carry

    lax.fori_loop(0, nch, chunk, 0)
    plsc.subcore_barrier()
    pltpu.sync_copy(acc.at[pl.ds(row0, RPT)], out.at[pl.ds(c * NPAD + row0, RPT)])

  params = {}
  if width < 128:
    params["compiler_params"] = pltpu.CompilerParams(use_tc_tiling_on_sc=False)
  return pl.kernel(
      body,
      out_type=jax.ShapeDtypeStruct((NC * NPAD, width), jnp.float32),
      mesh=_mesh(),
      scratch_types=[
          pltpu.VMEM((CH,), jnp.int32),
          pltpu.VMEM((CH,), jnp.int32),
          pltpu.VMEM((CH, width), jnp.float32),
          pltpu.VMEM_SHARED((NPAD, width), jnp.float32),
          pltpu.SemaphoreType.DMA,
          pltpu.SemaphoreType.DMA,
      ],
      **params,
  )


_BR = 512
_GRID = NPAD // _BR


def _scale_body(degp_ref, x_ref, dinv_ref, xst_ref):
  d = degp_ref[0] + degp_ref[1] - 1.0
  dinv = jnp.where(d >= 1.0, lax.rsqrt(d), 0.0)
  dinv_ref[...] = dinv
  xs = x_ref[...] * dinv
  xst_ref[0] = xs[:, :F]
  xst_ref[1] = xs[:, F:]


def _tc_scale(degp, x_p):
  return pl.pallas_call(
      _scale_body,
      grid=(_GRID,),
      in_specs=[
          pl.BlockSpec((2, _BR, 1), lambda i: (0, i, 0)),
          pl.BlockSpec((_BR, 256), lambda i: (i, 0)),
      ],
      out_specs=[
          pl.BlockSpec((_BR, 1), lambda i: (i, 0)),
          pl.BlockSpec((2, _BR, F), lambda i: (0, i, 0)),
      ],
      out_shape=[
          jax.ShapeDtypeStruct((NPAD, 1), jnp.float32),
          jax.ShapeDtypeStruct((2, NPAD, F), jnp.float32),
      ],
  )(degp, x_p)


def _mm1_body(yt_ref, dinv_ref, w_ref, b_ref, out_ref):
  h = jnp.dot(yt_ref[0], w_ref[:F, :], preferred_element_type=jnp.float32)
  h += jnp.dot(yt_ref[1], w_ref[F:, :], preferred_element_type=jnp.float32)
  dinv = dinv_ref[...]
  hs = dinv * jax.nn.relu(dinv * h + b_ref[...])
  for j in range(4):
    out_ref[j] = hs[:, j * F:(j + 1) * F]


def _tc_mm1(y1t, dinv, W1, b1):
  return pl.pallas_call(
      _mm1_body,
      grid=(_GRID,),
      in_specs=[
          pl.BlockSpec((2, _BR, F), lambda i: (0, i, 0)),
          pl.BlockSpec((_BR, 1), lambda i: (i, 0)),
          pl.BlockSpec((256, 512), lambda i: (0, 0)),
          pl.BlockSpec((1, 512), lambda i: (0, 0)),
      ],
      out_specs=pl.BlockSpec((4, _BR, F), lambda i: (0, i, 0)),
      out_shape=jax.ShapeDtypeStruct((4, NPAD, F), jnp.float32),
  )(y1t, dinv, W1, b1)


def _mm2_body(yt_ref, dinv_ref, w2_ref, b2_ref, w3_ref, zs_ref):
  h = jnp.dot(yt_ref[0], w2_ref[:F, :], preferred_element_type=jnp.float32)
  for j in range(1, 4):
    h += jnp.dot(yt_ref[j], w2_ref[j * F:(j + 1) * F, :],
                 preferred_element_type=jnp.float32)
  dinv = dinv_ref[...]
  t = jax.nn.relu(dinv * h + b2_ref[...])
  z = jnp.dot(t, w3_ref[...], preferred_element_type=jnp.float32)
  zs_ref[...] = dinv * z


def _tc_mm2(y2t, dinv, W2, b2, W3):
  return pl.pallas_call(
      _mm2_body,
      grid=(_GRID,),
      in_specs=[
          pl.BlockSpec((4, _BR, F), lambda i: (0, i, 0)),
          pl.BlockSpec((_BR, 1), lambda i: (i, 0)),
          pl.BlockSpec((512, 512), lambda i: (0, 0)),
          pl.BlockSpec((1, 512), lambda i: (0, 0)),
          pl.BlockSpec((512, 1), lambda i: (0, 0)),
      ],
      out_specs=pl.BlockSpec((_BR, 1), lambda i: (i, 0)),
      out_shape=jax.ShapeDtypeStruct((NPAD, 1), jnp.float32),
  )(y2t, dinv, W2, b2, W3)


def _pool_body(p_ref, zs_ref, dinv_ref, batch_ref, b3_ref, out_ref):
  v = (p_ref[0] + p_ref[1] - zs_ref[...]) * dinv_ref[...]
  rows = lax.broadcasted_iota(jnp.int32, (NPAD, 1), 0)
  v = jnp.where(rows < N, v, 0.0)
  gids = lax.broadcasted_iota(jnp.int32, (G, 1), 0)
  mask = (batch_ref[...] == gids).astype(jnp.float32)
  sums = jnp.dot(mask, v, preferred_element_type=jnp.float32)
  counts = jnp.sum(mask, axis=1, keepdims=True)
  out_ref[...] = sums / jnp.maximum(counts, 1.0) + b3_ref[...]


def _tc_pool(y3p, zs, dinv, batch2d, b3):
  return pl.pallas_call(
      _pool_body,
      out_shape=jax.ShapeDtypeStruct((G, 1), jnp.float32),
  )(y3p, zs, dinv, batch2d, b3)


_wide_agg1 = _make_wide_agg(1)
_wide_agg2 = _make_wide_agg(2)
_scalar_agg_ones = _make_scalar_agg(gather=False, width=16)
_scalar_agg = _make_scalar_agg(gather=True, width=16)


@jax.jit
def kernel(x, edge_index, batch, W1, b1, W2, b2, W3, b3):
  order = jnp.argsort(edge_index[0])
  src = edge_index[0][order]
  dst = edge_index[1][order]
  pad = EPAD - E
  src_p = jnp.concatenate([src, jnp.zeros((pad,), jnp.int32)])
  trash = N + (jnp.arange(pad, dtype=jnp.int32) % (NPAD - N - 8))
  dst_p = jnp.concatenate([dst, trash])

  x_p = jnp.pad(x, ((0, NPAD - N), (0, 0)))
  ones_t = jnp.ones((NPAD, 16), jnp.float32)

  src3 = src_p.reshape(NS, NCHW, CHW)
  srcpad = jnp.concatenate([src3, jnp.zeros((NS, 1, CHW), jnp.int32)], axis=1)
  srcoff4 = srcpad[None] + (jnp.arange(4, dtype=jnp.int32) * NPAD)[:, None, None, None]

  degp = _scalar_agg_ones(ones_t, src_p, dst_p).reshape(2, NPAD, 16)
  dinv, xs_t = _tc_scale(degp[:, :, :1], x_p)

  y1t = _wide_agg1(xs_t.reshape(2 * NPAD, F), srcoff4[:2], dst_p)
  h1st = _tc_mm1(y1t.reshape(2, NPAD, F), dinv, W1, b1.reshape(1, 512))

  y2t = _wide_agg2(h1st.reshape(4 * NPAD, F), srcoff4, dst_p)
  zs = _tc_mm2(y2t.reshape(4, NPAD, F), dinv, W2, b2.reshape(1, 512),
               W3.reshape(512, 1))

  zs16 = jnp.pad(zs, ((0, 0), (0, 15)))
  y3p = _scalar_agg(zs16, src_p, dst_p).reshape(2, NPAD, 16)[:, :, :1]

  batch_p = jnp.concatenate(
      [batch, jnp.full((NPAD - N,), 100, jnp.int32)]).reshape(1, NPAD)
  out = _tc_pool(y3p, zs, dinv, batch_p, b3.reshape(1, 1))
  return out[:, 0]


# speedup vs baseline: 1.7342x; 1.7342x over previous
import functools

import jax
import jax.numpy as jnp
from jax import lax
from jax.experimental import pallas as pl
from jax.experimental.pallas import tpu as pltpu
from jax.experimental.pallas import tpu_sc as plsc

N = 10000
NPAD = 10240
E = 160000
EPAD = 163840
G = 8
F = 128
CH = 128
NC, NS = 2, 16
RPT = NPAD // NS

_mesh = lambda: plsc.VectorSubcoreMesh(
    core_axis_name="c", subcore_axis_name="s", num_cores=NC, num_subcores=NS)


CHW = 64
NCHW = EPAD // NS // CHW
WF = 64


def _make_wide_agg(npass):
  nf = NC * npass

  def body(table, srcp3, dstf, out, gidx, dv0, dv1, rows0, rows1, stab, acc,
           gsem0, gsem1, ssem0, ssem1):
    c = lax.axis_index("c")
    s = lax.axis_index("s")
    row0 = s * RPT
    pltpu.sync_copy(srcp3.at[s], gidx)
    for p in range(npass):
      f = c * npass + p
      pltpu.sync_copy(table.at[pl.ds(f * NPAD + row0, RPT)],
                      stab.at[pl.ds(row0, RPT)])
      pltpu.sync_copy(table.at[pl.ds(f * NPAD + row0, RPT)],
                      acc.at[pl.ds(row0, RPT)])
      plsc.subcore_barrier()

      pltpu.async_copy(stab.at[gidx.at[0]], rows0, gsem0)

      def chunk2(i, carry):
        j0 = 2 * i
        base = s * (NCHW * CHW)
        pltpu.sync_copy(dstf.at[pl.ds(base + j0 * CHW, CHW)], dv0)
        pltpu.sync_copy(dstf.at[pl.ds(base + (j0 + 1) * CHW, CHW)], dv1)
        pltpu.make_async_copy(stab.at[gidx.at[j0]], rows0, gsem0).wait()
        pltpu.async_copy(stab.at[gidx.at[j0 + 1]], rows1, gsem1)
        sd0 = pltpu.async_copy(rows0, acc.at[dv0], ssem0, add=True)
        pltpu.make_async_copy(stab.at[gidx.at[j0 + 1]], rows1, gsem1).wait()
        sd0.wait()
        pltpu.async_copy(stab.at[gidx.at[j0 + 2]], rows0, gsem0)
        sd1 = pltpu.async_copy(rows1, acc.at[dv1], ssem1, add=True)
        sd1.wait()
        return carry

      lax.fori_loop(0, NCHW // 2, chunk2, 0)
      pltpu.make_async_copy(stab.at[gidx.at[NCHW]], rows0, gsem0).wait()
      plsc.subcore_barrier()
      pltpu.sync_copy(acc.at[pl.ds(row0, RPT)], out.at[pl.ds(f * NPAD + row0, RPT)])
      plsc.subcore_barrier()

  return pl.kernel(
      body,
      out_type=jax.ShapeDtypeStruct((nf * NPAD, WF), jnp.float32),
      mesh=_mesh(),
      compiler_params=pltpu.CompilerParams(use_tc_tiling_on_sc=False),
      scratch_types=[
          pltpu.VMEM((NCHW + 1, CHW), jnp.int32),
          pltpu.VMEM((CHW,), jnp.int32),
          pltpu.VMEM((CHW,), jnp.int32),
          pltpu.VMEM((CHW, WF), jnp.float32),
          pltpu.VMEM((CHW, WF), jnp.float32),
          pltpu.VMEM_SHARED((NPAD, WF), jnp.float32),
          pltpu.VMEM_SHARED((NPAD, WF), jnp.float32),
          pltpu.SemaphoreType.DMA,
          pltpu.SemaphoreType.DMA,
          pltpu.SemaphoreType.DMA,
          pltpu.SemaphoreType.DMA,
      ],
  )


def _make_scalar_agg(gather, width=F):
  ept = EPAD // (NC * NS)
  nch = ept // CH

  def body(table, srcp, dstp, out, src_v, dst_v, rows_v, acc, gsem, ssem):
    c = lax.axis_index("c")
    s = lax.axis_index("s")
    row0 = s * RPT
    wid = s * NC + c
    pltpu.sync_copy(table.at[pl.ds(row0, RPT)], acc.at[pl.ds(row0, RPT)])
    if not gather:
      pltpu.sync_copy(table.at[pl.ds(0, CH)], rows_v)
    plsc.subcore_barrier()

    def chunk(i, carry):
      off = wid * ept + i * CH
      pltpu.sync_copy(srcp.at[pl.ds(off, CH)], src_v)
      pltpu.sync_copy(dstp.at[pl.ds(off, CH)], dst_v)
      if gather:
        pltpu.async_copy(table.at[src_v], rows_v, gsem).wait()
      pltpu.async_copy(rows_v, acc.at[dst_v], ssem, add=True).wait()
      return carry

    lax.fori_loop(0, nch, chunk, 0)
    plsc.subcore_barrier()
    pltpu.sync_copy(acc.at[pl.ds(row0, RPT)], out.at[pl.ds(c * NPAD + row0, RPT)])

  params = {}
  if width < 128:
    params["compiler_params"] = pltpu.CompilerParams(use_tc_tiling_on_sc=False)
  return pl.kernel(
      body,
      out_type=jax.ShapeDtypeStruct((NC * NPAD, width), jnp.float32),
      mesh=_mesh(),
      scratch_types=[
          pltpu.VMEM((CH,), jnp.int32),
          pltpu.VMEM((CH,), jnp.int32),
          pltpu.VMEM((CH, width), jnp.float32),
          pltpu.VMEM_SHARED((NPAD, width), jnp.float32),
          pltpu.SemaphoreType.DMA,
          pltpu.SemaphoreType.DMA,
      ],
      **params,
  )


_BR = 512
_GRID = NPAD // _BR


def _scale_body(degp_ref, x_ref, dinv_ref, xst_ref):
  d = degp_ref[0] + degp_ref[1] - 1.0
  dinv = jnp.where(d >= 1.0, lax.rsqrt(d), 0.0)
  dinv_ref[...] = dinv
  xs = x_ref[...] * dinv
  for j in range(4):
    xst_ref[j] = xs[:, j * WF:(j + 1) * WF]


def _tc_scale(degp, x_p):
  return pl.pallas_call(
      _scale_body,
      grid=(_GRID,),
      in_specs=[
          pl.BlockSpec((2, _BR, 1), lambda i: (0, i, 0)),
          pl.BlockSpec((_BR, 256), lambda i: (i, 0)),
      ],
      out_specs=[
          pl.BlockSpec((_BR, 1), lambda i: (i, 0)),
          pl.BlockSpec((4, _BR, WF), lambda i: (0, i, 0)),
      ],
      out_shape=[
          jax.ShapeDtypeStruct((NPAD, 1), jnp.float32),
          jax.ShapeDtypeStruct((4, NPAD, WF), jnp.float32),
      ],
  )(degp, x_p)


def _mm1_body(yt_ref, dinv_ref, w_ref, b_ref, out_ref):
  h = jnp.dot(yt_ref[0], w_ref[:WF, :], preferred_element_type=jnp.float32)
  for j in range(1, 4):
    h += jnp.dot(yt_ref[j], w_ref[j * WF:(j + 1) * WF, :],
                 preferred_element_type=jnp.float32)
  dinv = dinv_ref[...]
  hs = dinv * jax.nn.relu(dinv * h + b_ref[...])
  for j in range(8):
    out_ref[j] = hs[:, j * WF:(j + 1) * WF]


def _tc_mm1(y1t, dinv, W1, b1):
  return pl.pallas_call(
      _mm1_body,
      grid=(_GRID,),
      in_specs=[
          pl.BlockSpec((4, _BR, WF), lambda i: (0, i, 0)),
          pl.BlockSpec((_BR, 1), lambda i: (i, 0)),
          pl.BlockSpec((256, 512), lambda i: (0, 0)),
          pl.BlockSpec((1, 512), lambda i: (0, 0)),
      ],
      out_specs=pl.BlockSpec((8, _BR, WF), lambda i: (0, i, 0)),
      out_shape=jax.ShapeDtypeStruct((8, NPAD, WF), jnp.float32),
  )(y1t, dinv, W1, b1)


def _mm2_body(yt_ref, dinv_ref, w2_ref, b2_ref, w3_ref, zs_ref):
  h = jnp.dot(yt_ref[0], w2_ref[:WF, :], preferred_element_type=jnp.float32)
  for j in range(1, 8):
    h += jnp.dot(yt_ref[j], w2_ref[j * WF:(j + 1) * WF, :],
                 preferred_element_type=jnp.float32)
  dinv = dinv_ref[...]
  t = jax.nn.relu(dinv * h + b2_ref[...])
  z = jnp.dot(t, w3_ref[...], preferred_element_type=jnp.float32)
  zs_ref[...] = dinv * z


def _tc_mm2(y2t, dinv, W2, b2, W3):
  return pl.pallas_call(
      _mm2_body,
      grid=(_GRID,),
      in_specs=[
          pl.BlockSpec((8, _BR, WF), lambda i: (0, i, 0)),
          pl.BlockSpec((_BR, 1), lambda i: (i, 0)),
          pl.BlockSpec((512, 512), lambda i: (0, 0)),
          pl.BlockSpec((1, 512), lambda i: (0, 0)),
          pl.BlockSpec((512, 1), lambda i: (0, 0)),
      ],
      out_specs=pl.BlockSpec((_BR, 1), lambda i: (i, 0)),
      out_shape=jax.ShapeDtypeStruct((NPAD, 1), jnp.float32),
  )(y2t, dinv, W2, b2, W3)


def _pool_body(p_ref, zs_ref, dinv_ref, batch_ref, b3_ref, out_ref):
  v = (p_ref[0] + p_ref[1] - zs_ref[...]) * dinv_ref[...]
  rows = lax.broadcasted_iota(jnp.int32, (NPAD, 1), 0)
  v = jnp.where(rows < N, v, 0.0)
  gids = lax.broadcasted_iota(jnp.int32, (G, 1), 0)
  mask = (batch_ref[...] == gids).astype(jnp.float32)
  sums = jnp.dot(mask, v, preferred_element_type=jnp.float32)
  counts = jnp.sum(mask, axis=1, keepdims=True)
  out_ref[...] = sums / jnp.maximum(counts, 1.0) + b3_ref[...]


def _tc_pool(y3p, zs, dinv, batch2d, b3):
  return pl.pallas_call(
      _pool_body,
      out_shape=jax.ShapeDtypeStruct((G, 1), jnp.float32),
  )(y3p, zs, dinv, batch2d, b3)


_wide_agg1 = _make_wide_agg(2)
_wide_agg2 = _make_wide_agg(4)
_scalar_agg_ones = _make_scalar_agg(gather=False, width=16)
_scalar_agg = _make_scalar_agg(gather=True, width=16)


@jax.jit
def kernel(x, edge_index, batch, W1, b1, W2, b2, W3, b3):
  src = edge_index[0]
  dst = edge_index[1]
  pad = EPAD - E
  src_p = jnp.concatenate([src, jnp.zeros((pad,), jnp.int32)])
  trash = N + (jnp.arange(pad, dtype=jnp.int32) % (NPAD - N - 8))
  dst_p = jnp.concatenate([dst, trash])

  x_p = jnp.pad(x, ((0, NPAD - N), (0, 0)))
  ones_t = jnp.ones((NPAD, 16), jnp.float32)

  src3 = src_p.reshape(NS, NCHW, CHW)
  srcp3 = jnp.concatenate([src3, jnp.zeros((NS, 1, CHW), jnp.int32)], axis=1)

  degp = _scalar_agg_ones(ones_t, src_p, dst_p).reshape(2, NPAD, 16)
  dinv, xs_t = _tc_scale(degp[:, :, :1], x_p)

  y1t = _wide_agg1(xs_t.reshape(4 * NPAD, WF), srcp3, dst_p)
  h1st = _tc_mm1(y1t.reshape(4, NPAD, WF), dinv, W1, b1.reshape(1, 512))

  y2t = _wide_agg2(h1st.reshape(8 * NPAD, WF), srcp3, dst_p)
  zs = _tc_mm2(y2t.reshape(8, NPAD, WF), dinv, W2, b2.reshape(1, 512),
               W3.reshape(512, 1))

  zs16 = jnp.pad(zs, ((0, 0), (0, 15)))
  y3p = _scalar_agg(zs16, src_p, dst_p).reshape(2, NPAD, 16)[:, :, :1]

  batch_p = jnp.concatenate(
      [batch, jnp.full((NPAD - N,), 100, jnp.int32)]).reshape(1, NPAD)
  out = _tc_pool(y3p, zs, dinv, batch_p, b3.reshape(1, 1))
  return out[:, 0]


# speedup vs baseline: 2.3345x; 1.3461x over previous
import functools

import jax
import jax.numpy as jnp
from jax import lax
from jax.experimental import pallas as pl
from jax.experimental.pallas import tpu as pltpu
from jax.experimental.pallas import tpu_sc as plsc

N = 10000
NPAD = 10240
E = 160000
EPAD = 163840
G = 8
F = 128
CH = 128
NC, NS = 2, 16
RPT = NPAD // NS

_mesh = lambda: plsc.VectorSubcoreMesh(
    core_axis_name="c", subcore_axis_name="s", num_cores=NC, num_subcores=NS)


CHW = 64
NCHW = EPAD // NS // CHW
WF = 64


def _make_wide_agg(npass):
  nf = NC * npass

  def body(table, combo3, out, combo, gv0, gv1, dv0, dv1, rows0, rows1,
           stab, acc, gsem0, gsem1, ssem0, ssem1):
    c = lax.axis_index("c")
    s = lax.axis_index("s")
    row0 = s * RPT
    pltpu.sync_copy(combo3.at[s], combo)

    def unpack(j, gv, dv):
      for k in range(CHW // 16):
        cw = combo[j, pl.ds(k * 16, 16)]
        gv[pl.ds(k * 16, 16)] = cw & 0xFFFF
        dv[pl.ds(k * 16, 16)] = lax.shift_right_logical(cw, 16)

    for p in range(npass):
      f = c * npass + p
      pltpu.sync_copy(table.at[pl.ds(f * NPAD + row0, RPT)],
                      stab.at[pl.ds(row0, RPT)])
      pltpu.sync_copy(table.at[pl.ds(f * NPAD + row0, RPT)],
                      acc.at[pl.ds(row0, RPT)])
      plsc.subcore_barrier()

      unpack(0, gv0, dv0)
      pltpu.async_copy(stab.at[gv0], rows0, gsem0)

      def chunk2(i, carry):
        j0 = 2 * i
        unpack(j0 + 1, gv1, dv1)
        pltpu.make_async_copy(stab.at[gv0], rows0, gsem0).wait()
        pltpu.async_copy(stab.at[gv1], rows1, gsem1)
        sd0 = pltpu.async_copy(rows0, acc.at[dv0], ssem0, add=True)
        pltpu.make_async_copy(stab.at[gv1], rows1, gsem1).wait()
        sd0.wait()
        unpack(j0 + 2, gv0, dv0)
        pltpu.async_copy(stab.at[gv0], rows0, gsem0)
        sd1 = pltpu.async_copy(rows1, acc.at[dv1], ssem1, add=True)
        sd1.wait()
        return carry

      lax.fori_loop(0, NCHW // 2, chunk2, 0)
      pltpu.make_async_copy(stab.at[gv0], rows0, gsem0).wait()
      plsc.subcore_barrier()
      pltpu.sync_copy(acc.at[pl.ds(row0, RPT)], out.at[pl.ds(f * NPAD + row0, RPT)])
      plsc.subcore_barrier()

  return pl.kernel(
      body,
      out_type=jax.ShapeDtypeStruct((nf * NPAD, WF), jnp.float32),
      mesh=_mesh(),
      compiler_params=pltpu.CompilerParams(use_tc_tiling_on_sc=False),
      scratch_types=[
          pltpu.VMEM((NCHW + 1, CHW), jnp.int32),
          pltpu.VMEM((CHW,), jnp.int32),
          pltpu.VMEM((CHW,), jnp.int32),
          pltpu.VMEM((CHW,), jnp.int32),
          pltpu.VMEM((CHW,), jnp.int32),
          pltpu.VMEM((CHW, WF), jnp.float32),
          pltpu.VMEM((CHW, WF), jnp.float32),
          pltpu.VMEM_SHARED((NPAD, WF), jnp.float32),
          pltpu.VMEM_SHARED((NPAD, WF), jnp.float32),
          pltpu.SemaphoreType.DMA,
          pltpu.SemaphoreType.DMA,
          pltpu.SemaphoreType.DMA,
          pltpu.SemaphoreType.DMA,
      ],
  )


def _make_scalar_agg(gather, width=F):
  ept = EPAD // (NC * NS)
  nch = ept // CH

  def body(table, srcp, dstp, out, src_v, dst_v, rows_v, acc, gsem, ssem):
    c = lax.axis_index("c")
    s = lax.axis_index("s")
    row0 = s * RPT
    wid = s * NC + c
    pltpu.sync_copy(table.at[pl.ds(row0, RPT)], acc.at[pl.ds(row0, RPT)])
    if not gather:
      pltpu.sync_copy(table.at[pl.ds(0, CH)], rows_v)
    plsc.subcore_barrier()

    def chunk(i, carry):
      off = wid * ept + i * CH
      pltpu.sync_copy(srcp.at[pl.ds(off, CH)], src_v)
      pltpu.sync_copy(dstp.at[pl.ds(off, CH)], dst_v)
      if gather:
        pltpu.async_copy(table.at[src_v], rows_v, gsem).wait()
      pltpu.async_copy(rows_v, acc.at[dst_v], ssem, add=True).wait()
      return carry

    lax.fori_loop(0, nch, chunk, 0)
    plsc.subcore_barrier()
    pltpu.sync_copy(acc.at[pl.ds(row0, RPT)], out.at[pl.ds(c * NPAD + row0, RPT)])

  params = {}
  if width < 128:
    params["compiler_params"] = pltpu.CompilerParams(use_tc_tiling_on_sc=False)
  return pl.kernel(
      body,
      out_type=jax.ShapeDtypeStruct((NC * NPAD, width), jnp.float32),
      mesh=_mesh(),
      scratch_types=[
          pltpu.VMEM((CH,), jnp.int32),
          pltpu.VMEM((CH,), jnp.int32),
          pltpu.VMEM((CH, width), jnp.float32),
          pltpu.VMEM_SHARED((NPAD, width), jnp.float32),
          pltpu.SemaphoreType.DMA,
          pltpu.SemaphoreType.DMA,
      ],
      **params,
  )


_BR = 512
_GRID = NPAD // _BR


def _scale_body(degp_ref, x_ref, dinv_ref, xst_ref):
  d = degp_ref[0] + degp_ref[1] - 1.0
  dinv = jnp.where(d >= 1.0, lax.rsqrt(d), 0.0)
  dinv_ref[...] = dinv
  xs = x_ref[...] * dinv
  for j in range(4):
    xst_ref[j] = xs[:, j * WF:(j + 1) * WF]


def _tc_scale(degp, x_p):
  return pl.pallas_call(
      _scale_body,
      grid=(_GRID,),
      in_specs=[
          pl.BlockSpec((2, _BR, 1), lambda i: (0, i, 0)),
          pl.BlockSpec((_BR, 256), lambda i: (i, 0)),
      ],
      out_specs=[
          pl.BlockSpec((_BR, 1), lambda i: (i, 0)),
          pl.BlockSpec((4, _BR, WF), lambda i: (0, i, 0)),
      ],
      out_shape=[
          jax.ShapeDtypeStruct((NPAD, 1), jnp.float32),
          jax.ShapeDtypeStruct((4, NPAD, WF), jnp.float32),
      ],
  )(degp, x_p)


def _mm1_body(yt_ref, dinv_ref, w_ref, b_ref, out_ref):
  h = jnp.dot(yt_ref[0], w_ref[:WF, :], preferred_element_type=jnp.float32)
  for j in range(1, 4):
    h += jnp.dot(yt_ref[j], w_ref[j * WF:(j + 1) * WF, :],
                 preferred_element_type=jnp.float32)
  dinv = dinv_ref[...]
  hs = dinv * jax.nn.relu(dinv * h + b_ref[...])
  for j in range(8):
    out_ref[j] = hs[:, j * WF:(j + 1) * WF]


def _tc_mm1(y1t, dinv, W1, b1):
  return pl.pallas_call(
      _mm1_body,
      grid=(_GRID,),
      in_specs=[
          pl.BlockSpec((4, _BR, WF), lambda i: (0, i, 0)),
          pl.BlockSpec((_BR, 1), lambda i: (i, 0)),
          pl.BlockSpec((256, 512), lambda i: (0, 0)),
          pl.BlockSpec((1, 512), lambda i: (0, 0)),
      ],
      out_specs=pl.BlockSpec((8, _BR, WF), lambda i: (0, i, 0)),
      out_shape=jax.ShapeDtypeStruct((8, NPAD, WF), jnp.float32),
  )(y1t, dinv, W1, b1)


def _mm2_body(yt_ref, dinv_ref, w2_ref, b2_ref, w3_ref, zs_ref):
  h = jnp.dot(yt_ref[0], w2_ref[:WF, :], preferred_element_type=jnp.float32)
  for j in range(1, 8):
    h += jnp.dot(yt_ref[j], w2_ref[j * WF:(j + 1) * WF, :],
                 preferred_element_type=jnp.float32)
  dinv = dinv_ref[...]
  t = jax.nn.relu(dinv * h + b2_ref[...])
  z = jnp.dot(t, w3_ref[...], preferred_element_type=jnp.float32)
  zs_ref[...] = dinv * z


def _tc_mm2(y2t, dinv, W2, b2, W3):
  return pl.pallas_call(
      _mm2_body,
      grid=(_GRID,),
      in_specs=[
          pl.BlockSpec((8, _BR, WF), lambda i: (0, i, 0)),
          pl.BlockSpec((_BR, 1), lambda i: (i, 0)),
          pl.BlockSpec((512, 512), lambda i: (0, 0)),
          pl.BlockSpec((1, 512), lambda i: (0, 0)),
          pl.BlockSpec((512, 1), lambda i: (0, 0)),
      ],
      out_specs=pl.BlockSpec((_BR, 1), lambda i: (i, 0)),
      out_shape=jax.ShapeDtypeStruct((NPAD, 1), jnp.float32),
  )(y2t, dinv, W2, b2, W3)


def _pool_body(p_ref, zs_ref, dinv_ref, batch_ref, b3_ref, out_ref):
  v = (p_ref[0] + p_ref[1] - zs_ref[...]) * dinv_ref[...]
  rows = lax.broadcasted_iota(jnp.int32, (NPAD, 1), 0)
  v = jnp.where(rows < N, v, 0.0)
  gids = lax.broadcasted_iota(jnp.int32, (G, 1), 0)
  mask = (batch_ref[...] == gids).astype(jnp.float32)
  sums = jnp.dot(mask, v, preferred_element_type=jnp.float32)
  counts = jnp.sum(mask, axis=1, keepdims=True)
  out_ref[...] = sums / jnp.maximum(counts, 1.0) + b3_ref[...]


def _tc_pool(y3p, zs, dinv, batch2d, b3):
  return pl.pallas_call(
      _pool_body,
      out_shape=jax.ShapeDtypeStruct((G, 1), jnp.float32),
  )(y3p, zs, dinv, batch2d, b3)


_wide_agg1 = _make_wide_agg(2)
_wide_agg2 = _make_wide_agg(4)
_scalar_agg_ones = _make_scalar_agg(gather=False, width=16)
_scalar_agg = _make_scalar_agg(gather=True, width=16)


@jax.jit
def kernel(x, edge_index, batch, W1, b1, W2, b2, W3, b3):
  src = edge_index[0]
  dst = edge_index[1]
  pad = EPAD - E
  src_p = jnp.concatenate([src, jnp.zeros((pad,), jnp.int32)])
  trash = N + (jnp.arange(pad, dtype=jnp.int32) % (NPAD - N - 8))
  dst_p = jnp.concatenate([dst, trash])

  x_p = jnp.pad(x, ((0, NPAD - N), (0, 0)))
  ones_t = jnp.ones((NPAD, 16), jnp.float32)

  src3 = src_p.reshape(NS, NCHW, CHW)
  srcp3 = jnp.concatenate([src3, jnp.zeros((NS, 1, CHW), jnp.int32)], axis=1)
  dst3 = dst_p.reshape(NS, NCHW, CHW)
  dstp3 = jnp.concatenate([dst3, jnp.zeros((NS, 1, CHW), jnp.int32)], axis=1)
  combo3 = jnp.left_shift(dstp3, 16) | srcp3

  degp = _scalar_agg_ones(ones_t, src_p, dst_p).reshape(2, NPAD, 16)
  dinv, xs_t = _tc_scale(degp[:, :, :1], x_p)

  y1t = _wide_agg1(xs_t.reshape(4 * NPAD, WF), combo3)
  h1st = _tc_mm1(y1t.reshape(4, NPAD, WF), dinv, W1, b1.reshape(1, 512))

  y2t = _wide_agg2(h1st.reshape(8 * NPAD, WF), combo3)
  zs = _tc_mm2(y2t.reshape(8, NPAD, WF), dinv, W2, b2.reshape(1, 512),
               W3.reshape(512, 1))

  zs16 = jnp.pad(zs, ((0, 0), (0, 15)))
  y3p = _scalar_agg(zs16, src_p, dst_p).reshape(2, NPAD, 16)[:, :, :1]

  batch_p = jnp.concatenate(
      [batch, jnp.full((NPAD - N,), 100, jnp.int32)]).reshape(1, NPAD)
  out = _tc_pool(y3p, zs, dinv, batch_p, b3.reshape(1, 1))
  return out[:, 0]


# speedup vs baseline: 2.6567x; 1.1380x over previous
import functools

import jax
import jax.numpy as jnp
from jax import lax
from jax.experimental import pallas as pl
from jax.experimental.pallas import tpu as pltpu
from jax.experimental.pallas import tpu_sc as plsc

N = 10000
NPAD = 10240
E = 160000
EPAD = 163840
G = 8
F = 128
CH = 128
NC, NS = 2, 16
RPT = NPAD // NS

_mesh = lambda: plsc.VectorSubcoreMesh(
    core_axis_name="c", subcore_axis_name="s", num_cores=NC, num_subcores=NS)


CHW = 64
NCHW = EPAD // NS // CHW
WF = 64


def _make_wide_agg(npass):
  nf = NC * npass

  def body(table, combo3, out, combo, gv0, gv1, dv0, dv1, rows0, rows1,
           stab, acc, gsem0, gsem1, ssem0, ssem1):
    c = lax.axis_index("c")
    s = lax.axis_index("s")
    row0 = s * RPT
    pltpu.sync_copy(combo3.at[s], combo)

    def unpack(j, gv, dv):
      for k in range(CHW // 16):
        cw = combo[j, pl.ds(k * 16, 16)]
        gv[pl.ds(k * 16, 16)] = cw & 0xFFFF
        dv[pl.ds(k * 16, 16)] = lax.shift_right_logical(cw, 16)

    for p in range(npass):
      f = c * npass + p
      pltpu.sync_copy(table.at[pl.ds(f * NPAD + row0, RPT)],
                      stab.at[pl.ds(row0, RPT)])
      pltpu.sync_copy(table.at[pl.ds(f * NPAD + row0, RPT)],
                      acc.at[pl.ds(row0, RPT)])
      plsc.subcore_barrier()

      unpack(0, gv0, dv0)
      pltpu.async_copy(stab.at[gv0], rows0, gsem0)

      def chunk2(i, carry):
        j0 = 2 * i
        unpack(j0 + 1, gv1, dv1)
        pltpu.make_async_copy(stab.at[gv0], rows0, gsem0).wait()
        pltpu.async_copy(stab.at[gv1], rows1, gsem1)
        sd0 = pltpu.async_copy(rows0, acc.at[dv0], ssem0, add=True)
        pltpu.make_async_copy(stab.at[gv1], rows1, gsem1).wait()
        sd0.wait()
        unpack(j0 + 2, gv0, dv0)
        pltpu.async_copy(stab.at[gv0], rows0, gsem0)
        sd1 = pltpu.async_copy(rows1, acc.at[dv1], ssem1, add=True)
        sd1.wait()
        return carry

      lax.fori_loop(0, NCHW // 2, chunk2, 0)
      pltpu.make_async_copy(stab.at[gv0], rows0, gsem0).wait()
      plsc.subcore_barrier()
      pltpu.sync_copy(acc.at[pl.ds(row0, RPT)], out.at[pl.ds(f * NPAD + row0, RPT)])
      plsc.subcore_barrier()

  return pl.kernel(
      body,
      out_type=jax.ShapeDtypeStruct((nf * NPAD, WF), jnp.float32),
      mesh=_mesh(),
      compiler_params=pltpu.CompilerParams(use_tc_tiling_on_sc=False),
      scratch_types=[
          pltpu.VMEM((NCHW + 1, CHW), jnp.int32),
          pltpu.VMEM((CHW,), jnp.int32),
          pltpu.VMEM((CHW,), jnp.int32),
          pltpu.VMEM((CHW,), jnp.int32),
          pltpu.VMEM((CHW,), jnp.int32),
          pltpu.VMEM((CHW, WF), jnp.float32),
          pltpu.VMEM((CHW, WF), jnp.float32),
          pltpu.VMEM_SHARED((NPAD, WF), jnp.float32),
          pltpu.VMEM_SHARED((NPAD, WF), jnp.float32),
          pltpu.SemaphoreType.DMA,
          pltpu.SemaphoreType.DMA,
          pltpu.SemaphoreType.DMA,
          pltpu.SemaphoreType.DMA,
      ],
  )


NCHS = EPAD // (NC * NS) // CH


def _make_scalar_agg(gather):
  W16 = 16

  def body(table, combo3s, out, combo, gv0, gv1, dv0, dv1, rows0, rows1,
           stab, acc, gsem0, gsem1, ssem0, ssem1):
    c = lax.axis_index("c")
    s = lax.axis_index("s")
    row0 = s * RPT
    wid = s * NC + c
    pltpu.sync_copy(combo3s.at[wid], combo)

    def unpack(j, gv, dv):
      for k in range(CH // 16):
        cw = combo[j, pl.ds(k * 16, 16)]
        gv[pl.ds(k * 16, 16)] = cw & 0xFFFF
        dv[pl.ds(k * 16, 16)] = lax.shift_right_logical(cw, 16)

    pltpu.sync_copy(table.at[pl.ds(row0, RPT)], acc.at[pl.ds(row0, RPT)])
    if gather:
      pltpu.sync_copy(table.at[pl.ds(row0, RPT)], stab.at[pl.ds(row0, RPT)])
    else:
      pltpu.sync_copy(table.at[pl.ds(0, CH)], rows0)
      pltpu.sync_copy(table.at[pl.ds(0, CH)], rows1)
    plsc.subcore_barrier()

    if gather:
      unpack(0, gv0, dv0)
      pltpu.async_copy(stab.at[gv0], rows0, gsem0)

      def chunk2(i, carry):
        j0 = 2 * i
        unpack(j0 + 1, gv1, dv1)
        pltpu.make_async_copy(stab.at[gv0], rows0, gsem0).wait()
        pltpu.async_copy(stab.at[gv1], rows1, gsem1)
        sd0 = pltpu.async_copy(rows0, acc.at[dv0], ssem0, add=True)
        pltpu.make_async_copy(stab.at[gv1], rows1, gsem1).wait()
        sd0.wait()
        unpack(j0 + 2, gv0, dv0)
        pltpu.async_copy(stab.at[gv0], rows0, gsem0)
        sd1 = pltpu.async_copy(rows1, acc.at[dv1], ssem1, add=True)
        sd1.wait()
        return carry

      lax.fori_loop(0, NCHS // 2, chunk2, 0)
      pltpu.make_async_copy(stab.at[gv0], rows0, gsem0).wait()
    else:
      unpack(0, gv0, dv0)
      sd0 = pltpu.async_copy(rows0, acc.at[dv0], ssem0, add=True)

      def chunk2(i, carry):
        j0 = 2 * i
        unpack(j0 + 1, gv1, dv1)
        pltpu.async_copy(rows1, acc.at[dv1], ssem1, add=True)
        pltpu.make_async_copy(rows0, acc.at[dv0], ssem0).wait()
        unpack(j0 + 2, gv0, dv0)
        pltpu.async_copy(rows0, acc.at[dv0], ssem0, add=True)
        pltpu.make_async_copy(rows1, acc.at[dv1], ssem1).wait()
        return carry

      lax.fori_loop(0, NCHS // 2, chunk2, 0)
      pltpu.make_async_copy(rows0, acc.at[dv0], ssem0).wait()

    plsc.subcore_barrier()
    pltpu.sync_copy(acc.at[pl.ds(row0, RPT)], out.at[pl.ds(c * NPAD + row0, RPT)])

  return pl.kernel(
      body,
      out_type=jax.ShapeDtypeStruct((NC * NPAD, W16), jnp.float32),
      mesh=_mesh(),
      compiler_params=pltpu.CompilerParams(use_tc_tiling_on_sc=False),
      scratch_types=[
          pltpu.VMEM((NCHS + 1, CH), jnp.int32),
          pltpu.VMEM((CH,), jnp.int32),
          pltpu.VMEM((CH,), jnp.int32),
          pltpu.VMEM((CH,), jnp.int32),
          pltpu.VMEM((CH,), jnp.int32),
          pltpu.VMEM((CH, W16), jnp.float32),
          pltpu.VMEM((CH, W16), jnp.float32),
          pltpu.VMEM_SHARED((NPAD, W16), jnp.float32),
          pltpu.VMEM_SHARED((NPAD, W16), jnp.float32),
          pltpu.SemaphoreType.DMA,
          pltpu.SemaphoreType.DMA,
          pltpu.SemaphoreType.DMA,
          pltpu.SemaphoreType.DMA,
      ],
  )


_BR = 512
_GRID = NPAD // _BR


def _scale_body(degp_ref, x_ref, dinv_ref, xst_ref):
  d = degp_ref[0] + degp_ref[1] - 1.0
  dinv = jnp.where(d >= 1.0, lax.rsqrt(d), 0.0)
  dinv_ref[...] = dinv
  xs = x_ref[...] * dinv
  for j in range(4):
    xst_ref[j] = xs[:, j * WF:(j + 1) * WF]


def _tc_scale(degp, x_p):
  return pl.pallas_call(
      _scale_body,
      grid=(_GRID,),
      in_specs=[
          pl.BlockSpec((2, _BR, 1), lambda i: (0, i, 0)),
          pl.BlockSpec((_BR, 256), lambda i: (i, 0)),
      ],
      out_specs=[
          pl.BlockSpec((_BR, 1), lambda i: (i, 0)),
          pl.BlockSpec((4, _BR, WF), lambda i: (0, i, 0)),
      ],
      out_shape=[
          jax.ShapeDtypeStruct((NPAD, 1), jnp.float32),
          jax.ShapeDtypeStruct((4, NPAD, WF), jnp.float32),
      ],
  )(degp, x_p)


def _mm1_body(yt_ref, dinv_ref, w_ref, b_ref, out_ref):
  h = jnp.dot(yt_ref[0], w_ref[:WF, :], preferred_element_type=jnp.float32)
  for j in range(1, 4):
    h += jnp.dot(yt_ref[j], w_ref[j * WF:(j + 1) * WF, :],
                 preferred_element_type=jnp.float32)
  dinv = dinv_ref[...]
  hs = dinv * jax.nn.relu(dinv * h + b_ref[...])
  for j in range(8):
    out_ref[j] = hs[:, j * WF:(j + 1) * WF]


def _tc_mm1(y1t, dinv, W1, b1):
  return pl.pallas_call(
      _mm1_body,
      grid=(_GRID,),
      in_specs=[
          pl.BlockSpec((4, _BR, WF), lambda i: (0, i, 0)),
          pl.BlockSpec((_BR, 1), lambda i: (i, 0)),
          pl.BlockSpec((256, 512), lambda i: (0, 0)),
          pl.BlockSpec((1, 512), lambda i: (0, 0)),
      ],
      out_specs=pl.BlockSpec((8, _BR, WF), lambda i: (0, i, 0)),
      out_shape=jax.ShapeDtypeStruct((8, NPAD, WF), jnp.float32),
  )(y1t, dinv, W1, b1)


def _mm2_body(yt_ref, dinv_ref, w2_ref, b2_ref, w3_ref, zs_ref):
  h = jnp.dot(yt_ref[0], w2_ref[:WF, :], preferred_element_type=jnp.float32)
  for j in range(1, 8):
    h += jnp.dot(yt_ref[j], w2_ref[j * WF:(j + 1) * WF, :],
                 preferred_element_type=jnp.float32)
  dinv = dinv_ref[...]
  t = jax.nn.relu(dinv * h + b2_ref[...])
  z = jnp.dot(t, w3_ref[...], preferred_element_type=jnp.float32)
  zs_ref[...] = dinv * z


def _tc_mm2(y2t, dinv, W2, b2, W3):
  return pl.pallas_call(
      _mm2_body,
      grid=(_GRID,),
      in_specs=[
          pl.BlockSpec((8, _BR, WF), lambda i: (0, i, 0)),
          pl.BlockSpec((_BR, 1), lambda i: (i, 0)),
          pl.BlockSpec((512, 512), lambda i: (0, 0)),
          pl.BlockSpec((1, 512), lambda i: (0, 0)),
          pl.BlockSpec((512, 1), lambda i: (0, 0)),
      ],
      out_specs=pl.BlockSpec((_BR, 1), lambda i: (i, 0)),
      out_shape=jax.ShapeDtypeStruct((NPAD, 1), jnp.float32),
  )(y2t, dinv, W2, b2, W3)


def _pool_body(p_ref, zs_ref, dinv_ref, batch_ref, b3_ref, out_ref):
  v = (p_ref[0] + p_ref[1] - zs_ref[...]) * dinv_ref[...]
  rows = lax.broadcasted_iota(jnp.int32, (NPAD, 1), 0)
  v = jnp.where(rows < N, v, 0.0)
  gids = lax.broadcasted_iota(jnp.int32, (G, 1), 0)
  mask = (batch_ref[...] == gids).astype(jnp.float32)
  sums = jnp.dot(mask, v, preferred_element_type=jnp.float32)
  counts = jnp.sum(mask, axis=1, keepdims=True)
  out_ref[...] = sums / jnp.maximum(counts, 1.0) + b3_ref[...]


def _tc_pool(y3p, zs, dinv, batch2d, b3):
  return pl.pallas_call(
      _pool_body,
      out_shape=jax.ShapeDtypeStruct((G, 1), jnp.float32),
  )(y3p, zs, dinv, batch2d, b3)


_wide_agg1 = _make_wide_agg(2)
_wide_agg2 = _make_wide_agg(4)
_scalar_agg_ones = _make_scalar_agg(gather=False)
_scalar_agg = _make_scalar_agg(gather=True)


@jax.jit
def kernel(x, edge_index, batch, W1, b1, W2, b2, W3, b3):
  src = edge_index[0]
  dst = edge_index[1]
  pad = EPAD - E
  src_p = jnp.concatenate([src, jnp.zeros((pad,), jnp.int32)])
  trash = N + (jnp.arange(pad, dtype=jnp.int32) % (NPAD - N - 8))
  dst_p = jnp.concatenate([dst, trash])

  x_p = jnp.pad(x, ((0, NPAD - N), (0, 0)))
  ones_t = jnp.ones((NPAD, 16), jnp.float32)

  src3 = src_p.reshape(NS, NCHW, CHW)
  srcp3 = jnp.concatenate([src3, jnp.zeros((NS, 1, CHW), jnp.int32)], axis=1)
  dst3 = dst_p.reshape(NS, NCHW, CHW)
  dstp3 = jnp.concatenate([dst3, jnp.zeros((NS, 1, CHW), jnp.int32)], axis=1)
  combo3 = jnp.left_shift(dstp3, 16) | srcp3
  src_s = jnp.concatenate(
      [src_p.reshape(NC * NS, NCHS, CH),
       jnp.zeros((NC * NS, 1, CH), jnp.int32)], axis=1)
  trash_s = N + (jnp.arange(CH, dtype=jnp.int32) % (NPAD - N - 8))
  dst_s = jnp.concatenate(
      [dst_p.reshape(NC * NS, NCHS, CH),
       jnp.broadcast_to(trash_s, (NC * NS, 1, CH))], axis=1)
  combo3s = jnp.left_shift(dst_s, 16) | src_s

  degp = _scalar_agg_ones(ones_t, combo3s).reshape(2, NPAD, 16)
  dinv, xs_t = _tc_scale(degp[:, :, :1], x_p)

  y1t = _wide_agg1(xs_t.reshape(4 * NPAD, WF), combo3)
  h1st = _tc_mm1(y1t.reshape(4, NPAD, WF), dinv, W1, b1.reshape(1, 512))

  y2t = _wide_agg2(h1st.reshape(8 * NPAD, WF), combo3)
  zs = _tc_mm2(y2t.reshape(8, NPAD, WF), dinv, W2, b2.reshape(1, 512),
               W3.reshape(512, 1))

  zs16 = jnp.pad(zs, ((0, 0), (0, 15)))
  y3p = _scalar_agg(zs16, combo3s).reshape(2, NPAD, 16)[:, :, :1]

  batch_p = jnp.concatenate(
      [batch, jnp.full((NPAD - N,), 100, jnp.int32)]).reshape(1, NPAD)
  out = _tc_pool(y3p, zs, dinv, batch_p, b3.reshape(1, 1))
  return out[:, 0]


# speedup vs baseline: 2.7900x; 1.0502x over previous
import functools

import jax
import jax.numpy as jnp
from jax import lax
from jax.experimental import pallas as pl
from jax.experimental.pallas import tpu as pltpu
from jax.experimental.pallas import tpu_sc as plsc

N = 10000
NPAD = 10240
E = 160000
EPAD = 163840
G = 8
F = 128
CH = 128
NC, NS = 2, 16
RPT = NPAD // NS

_mesh = lambda: plsc.VectorSubcoreMesh(
    core_axis_name="c", subcore_axis_name="s", num_cores=NC, num_subcores=NS)


CHW = 64
NCHW = EPAD // NS // CHW
WF = 64


def _make_wide_agg(npass):
  nf = NC * npass

  def body(table, combo3, out, combo,
           gv0, gv1, gv2, gv3, dv0, dv1, dv2, dv3,
           rows0, rows1, rows2, rows3, stab, acc,
           gsem0, gsem1, gsem2, gsem3, ssem0, ssem1, ssem2, ssem3):
    c = lax.axis_index("c")
    s = lax.axis_index("s")
    row0 = s * RPT
    pltpu.sync_copy(combo3.at[s], combo)

    def unpack(j, gv, dv):
      for k in range(CHW // 16):
        cw = combo[j, pl.ds(k * 16, 16)]
        gv[pl.ds(k * 16, 16)] = cw & 0xFFFF
        dv[pl.ds(k * 16, 16)] = lax.shift_right_logical(cw, 16)

    for p in range(npass):
      f = c * npass + p
      pltpu.sync_copy(table.at[pl.ds(f * NPAD + row0, RPT)],
                      stab.at[pl.ds(row0, RPT)])
      pltpu.sync_copy(table.at[pl.ds(f * NPAD + row0, RPT)],
                      acc.at[pl.ds(row0, RPT)])
      plsc.subcore_barrier()

      unpack(0, gv0, dv0)
      pltpu.async_copy(stab.at[gv0], rows0, gsem0)
      unpack(1, gv1, dv1)
      pltpu.async_copy(stab.at[gv1], rows1, gsem1)

      def chunk4(i, carry):
        j = 4 * i
        pltpu.make_async_copy(stab.at[gv0], rows0, gsem0).wait()
        unpack(j + 2, gv2, dv2)
        pltpu.async_copy(stab.at[gv2], rows2, gsem2)
        sd0 = pltpu.async_copy(rows0, acc.at[dv0], ssem0, add=True)

        pltpu.make_async_copy(stab.at[gv1], rows1, gsem1).wait()
        unpack(j + 3, gv3, dv3)
        pltpu.async_copy(stab.at[gv3], rows3, gsem3)
        sd1 = pltpu.async_copy(rows1, acc.at[dv1], ssem1, add=True)

        pltpu.make_async_copy(stab.at[gv2], rows2, gsem2).wait()
        sd0.wait()
        unpack(j + 4, gv0, dv0)
        pltpu.async_copy(stab.at[gv0], rows0, gsem0)
        sd2 = pltpu.async_copy(rows2, acc.at[dv2], ssem2, add=True)

        pltpu.make_async_copy(stab.at[gv3], rows3, gsem3).wait()
        sd1.wait()
        unpack(j + 5, gv1, dv1)
        pltpu.async_copy(stab.at[gv1], rows1, gsem1)
        sd3 = pltpu.async_copy(rows3, acc.at[dv3], ssem3, add=True)

        sd2.wait()
        sd3.wait()
        return carry

      lax.fori_loop(0, NCHW // 4, chunk4, 0)
      pltpu.make_async_copy(stab.at[gv0], rows0, gsem0).wait()
      pltpu.make_async_copy(stab.at[gv1], rows1, gsem1).wait()
      plsc.subcore_barrier()
      pltpu.sync_copy(acc.at[pl.ds(row0, RPT)], out.at[pl.ds(f * NPAD + row0, RPT)])

  return pl.kernel(
      body,
      out_type=jax.ShapeDtypeStruct((nf * NPAD, WF), jnp.float32),
      mesh=_mesh(),
      compiler_params=pltpu.CompilerParams(use_tc_tiling_on_sc=False),
      scratch_types=[
          pltpu.VMEM((NCHW + 2, CHW), jnp.int32),
          pltpu.VMEM((CHW,), jnp.int32),
          pltpu.VMEM((CHW,), jnp.int32),
          pltpu.VMEM((CHW,), jnp.int32),
          pltpu.VMEM((CHW,), jnp.int32),
          pltpu.VMEM((CHW,), jnp.int32),
          pltpu.VMEM((CHW,), jnp.int32),
          pltpu.VMEM((CHW,), jnp.int32),
          pltpu.VMEM((CHW,), jnp.int32),
          pltpu.VMEM((CHW, WF), jnp.float32),
          pltpu.VMEM((CHW, WF), jnp.float32),
          pltpu.VMEM((CHW, WF), jnp.float32),
          pltpu.VMEM((CHW, WF), jnp.float32),
          pltpu.VMEM_SHARED((NPAD, WF), jnp.float32),
          pltpu.VMEM_SHARED((NPAD, WF), jnp.float32),
          pltpu.SemaphoreType.DMA,
          pltpu.SemaphoreType.DMA,
          pltpu.SemaphoreType.DMA,
          pltpu.SemaphoreType.DMA,
          pltpu.SemaphoreType.DMA,
          pltpu.SemaphoreType.DMA,
          pltpu.SemaphoreType.DMA,
          pltpu.SemaphoreType.DMA,
      ],
  )


NCHS = EPAD // (NC * NS) // CH


def _make_scalar_agg(gather):
  W16 = 16

  def body(table, combo3s, out, combo, gv0, gv1, dv0, dv1, rows0, rows1,
           stab, acc, gsem0, gsem1, ssem0, ssem1):
    c = lax.axis_index("c")
    s = lax.axis_index("s")
    row0 = s * RPT
    wid = s * NC + c
    pltpu.sync_copy(combo3s.at[wid], combo)

    def unpack(j, gv, dv):
      for k in range(CH // 16):
        cw = combo[j, pl.ds(k * 16, 16)]
        gv[pl.ds(k * 16, 16)] = cw & 0xFFFF
        dv[pl.ds(k * 16, 16)] = lax.shift_right_logical(cw, 16)

    pltpu.sync_copy(table.at[pl.ds(row0, RPT)], acc.at[pl.ds(row0, RPT)])
    if gather:
      pltpu.sync_copy(table.at[pl.ds(row0, RPT)], stab.at[pl.ds(row0, RPT)])
    else:
      pltpu.sync_copy(table.at[pl.ds(0, CH)], rows0)
      pltpu.sync_copy(table.at[pl.ds(0, CH)], rows1)
    plsc.subcore_barrier()

    if gather:
      unpack(0, gv0, dv0)
      pltpu.async_copy(stab.at[gv0], rows0, gsem0)

      def chunk2(i, carry):
        j0 = 2 * i
        unpack(j0 + 1, gv1, dv1)
        pltpu.make_async_copy(stab.at[gv0], rows0, gsem0).wait()
        pltpu.async_copy(stab.at[gv1], rows1, gsem1)
        sd0 = pltpu.async_copy(rows0, acc.at[dv0], ssem0, add=True)
        pltpu.make_async_copy(stab.at[gv1], rows1, gsem1).wait()
        sd0.wait()
        unpack(j0 + 2, gv0, dv0)
        pltpu.async_copy(stab.at[gv0], rows0, gsem0)
        sd1 = pltpu.async_copy(rows1, acc.at[dv1], ssem1, add=True)
        sd1.wait()
        return carry

      lax.fori_loop(0, NCHS // 2, chunk2, 0)
      pltpu.make_async_copy(stab.at[gv0], rows0, gsem0).wait()
    else:
      unpack(0, gv0, dv0)
      sd0 = pltpu.async_copy(rows0, acc.at[dv0], ssem0, add=True)

      def chunk2(i, carry):
        j0 = 2 * i
        unpack(j0 + 1, gv1, dv1)
        pltpu.async_copy(rows1, acc.at[dv1], ssem1, add=True)
        pltpu.make_async_copy(rows0, acc.at[dv0], ssem0).wait()
        unpack(j0 + 2, gv0, dv0)
        pltpu.async_copy(rows0, acc.at[dv0], ssem0, add=True)
        pltpu.make_async_copy(rows1, acc.at[dv1], ssem1).wait()
        return carry

      lax.fori_loop(0, NCHS // 2, chunk2, 0)
      pltpu.make_async_copy(rows0, acc.at[dv0], ssem0).wait()

    plsc.subcore_barrier()
    pltpu.sync_copy(acc.at[pl.ds(row0, RPT)], out.at[pl.ds(c * NPAD + row0, RPT)])

  return pl.kernel(
      body,
      out_type=jax.ShapeDtypeStruct((NC * NPAD, W16), jnp.float32),
      mesh=_mesh(),
      compiler_params=pltpu.CompilerParams(use_tc_tiling_on_sc=False),
      scratch_types=[
          pltpu.VMEM((NCHS + 1, CH), jnp.int32),
          pltpu.VMEM((CH,), jnp.int32),
          pltpu.VMEM((CH,), jnp.int32),
          pltpu.VMEM((CH,), jnp.int32),
          pltpu.VMEM((CH,), jnp.int32),
          pltpu.VMEM((CH, W16), jnp.float32),
          pltpu.VMEM((CH, W16), jnp.float32),
          pltpu.VMEM_SHARED((NPAD, W16), jnp.float32),
          pltpu.VMEM_SHARED((NPAD, W16), jnp.float32),
          pltpu.SemaphoreType.DMA,
          pltpu.SemaphoreType.DMA,
          pltpu.SemaphoreType.DMA,
          pltpu.SemaphoreType.DMA,
      ],
  )


_BR = 512
_GRID = NPAD // _BR


def _scale_body(degp_ref, x_ref, dinv_ref, xst_ref):
  d = degp_ref[0] + degp_ref[1] - 1.0
  dinv = jnp.where(d >= 1.0, lax.rsqrt(d), 0.0)
  dinv_ref[...] = dinv
  xs = x_ref[...] * dinv
  for j in range(4):
    xst_ref[j] = xs[:, j * WF:(j + 1) * WF]


def _tc_scale(degp, x_p):
  return pl.pallas_call(
      _scale_body,
      grid=(_GRID,),
      in_specs=[
          pl.BlockSpec((2, _BR, 1), lambda i: (0, i, 0)),
          pl.BlockSpec((_BR, 256), lambda i: (i, 0)),
      ],
      out_specs=[
          pl.BlockSpec((_BR, 1), lambda i: (i, 0)),
          pl.BlockSpec((4, _BR, WF), lambda i: (0, i, 0)),
      ],
      out_shape=[
          jax.ShapeDtypeStruct((NPAD, 1), jnp.float32),
          jax.ShapeDtypeStruct((4, NPAD, WF), jnp.float32),
      ],
  )(degp, x_p)


def _mm1_body(yt_ref, dinv_ref, w_ref, b_ref, out_ref):
  h = jnp.dot(yt_ref[0], w_ref[:WF, :], preferred_element_type=jnp.float32)
  for j in range(1, 4):
    h += jnp.dot(yt_ref[j], w_ref[j * WF:(j + 1) * WF, :],
                 preferred_element_type=jnp.float32)
  dinv = dinv_ref[...]
  hs = dinv * jax.nn.relu(dinv * h + b_ref[...])
  for j in range(8):
    out_ref[j] = hs[:, j * WF:(j + 1) * WF]


def _tc_mm1(y1t, dinv, W1, b1):
  return pl.pallas_call(
      _mm1_body,
      grid=(_GRID,),
      in_specs=[
          pl.BlockSpec((4, _BR, WF), lambda i: (0, i, 0)),
          pl.BlockSpec((_BR, 1), lambda i: (i, 0)),
          pl.BlockSpec((256, 512), lambda i: (0, 0)),
          pl.BlockSpec((1, 512), lambda i: (0, 0)),
      ],
      out_specs=pl.BlockSpec((8, _BR, WF), lambda i: (0, i, 0)),
      out_shape=jax.ShapeDtypeStruct((8, NPAD, WF), jnp.float32),
  )(y1t, dinv, W1, b1)


def _mm2_body(yt_ref, dinv_ref, w2_ref, b2_ref, w3_ref, zs_ref):
  h = jnp.dot(yt_ref[0], w2_ref[:WF, :], preferred_element_type=jnp.float32)
  for j in range(1, 8):
    h += jnp.dot(yt_ref[j], w2_ref[j * WF:(j + 1) * WF, :],
                 preferred_element_type=jnp.float32)
  dinv = dinv_ref[...]
  t = jax.nn.relu(dinv * h + b2_ref[...])
  z = jnp.dot(t, w3_ref[...], preferred_element_type=jnp.float32)
  zs_ref[...] = dinv * z


def _tc_mm2(y2t, dinv, W2, b2, W3):
  return pl.pallas_call(
      _mm2_body,
      grid=(_GRID,),
      in_specs=[
          pl.BlockSpec((8, _BR, WF), lambda i: (0, i, 0)),
          pl.BlockSpec((_BR, 1), lambda i: (i, 0)),
          pl.BlockSpec((512, 512), lambda i: (0, 0)),
          pl.BlockSpec((1, 512), lambda i: (0, 0)),
          pl.BlockSpec((512, 1), lambda i: (0, 0)),
      ],
      out_specs=pl.BlockSpec((_BR, 1), lambda i: (i, 0)),
      out_shape=jax.ShapeDtypeStruct((NPAD, 1), jnp.float32),
  )(y2t, dinv, W2, b2, W3)


def _pool_body(p_ref, zs_ref, dinv_ref, batch_ref, b3_ref, out_ref):
  v = (p_ref[0] + p_ref[1] - zs_ref[...]) * dinv_ref[...]
  rows = lax.broadcasted_iota(jnp.int32, (NPAD, 1), 0)
  v = jnp.where(rows < N, v, 0.0)
  gids = lax.broadcasted_iota(jnp.int32, (G, 1), 0)
  mask = (batch_ref[...] == gids).astype(jnp.float32)
  sums = jnp.dot(mask, v, preferred_element_type=jnp.float32)
  counts = jnp.sum(mask, axis=1, keepdims=True)
  out_ref[...] = sums / jnp.maximum(counts, 1.0) + b3_ref[...]


def _tc_pool(y3p, zs, dinv, batch2d, b3):
  return pl.pallas_call(
      _pool_body,
      out_shape=jax.ShapeDtypeStruct((G, 1), jnp.float32),
  )(y3p, zs, dinv, batch2d, b3)


_wide_agg1 = _make_wide_agg(2)
_wide_agg2 = _make_wide_agg(4)
_scalar_agg_ones = _make_scalar_agg(gather=False)
_scalar_agg = _make_scalar_agg(gather=True)


@jax.jit
def kernel(x, edge_index, batch, W1, b1, W2, b2, W3, b3):
  src = edge_index[0]
  dst = edge_index[1]
  pad = EPAD - E
  src_p = jnp.concatenate([src, jnp.zeros((pad,), jnp.int32)])
  trash = N + (jnp.arange(pad, dtype=jnp.int32) % (NPAD - N - 8))
  dst_p = jnp.concatenate([dst, trash])

  x_p = jnp.pad(x, ((0, NPAD - N), (0, 0)))
  ones_t = jnp.ones((NPAD, 16), jnp.float32)

  src3 = src_p.reshape(NS, NCHW, CHW)
  srcp3 = jnp.concatenate([src3, jnp.zeros((NS, 2, CHW), jnp.int32)], axis=1)
  dst3 = dst_p.reshape(NS, NCHW, CHW)
  dstp3 = jnp.concatenate([dst3, jnp.zeros((NS, 2, CHW), jnp.int32)], axis=1)
  combo3 = jnp.left_shift(dstp3, 16) | srcp3
  src_s = jnp.concatenate(
      [src_p.reshape(NC * NS, NCHS, CH),
       jnp.zeros((NC * NS, 1, CH), jnp.int32)], axis=1)
  trash_s = N + (jnp.arange(CH, dtype=jnp.int32) % (NPAD - N - 8))
  dst_s = jnp.concatenate(
      [dst_p.reshape(NC * NS, NCHS, CH),
       jnp.broadcast_to(trash_s, (NC * NS, 1, CH))], axis=1)
  combo3s = jnp.left_shift(dst_s, 16) | src_s

  degp = _scalar_agg_ones(ones_t, combo3s).reshape(2, NPAD, 16)
  dinv, xs_t = _tc_scale(degp[:, :, :1], x_p)

  y1t = _wide_agg1(xs_t.reshape(4 * NPAD, WF), combo3)
  h1st = _tc_mm1(y1t.reshape(4, NPAD, WF), dinv, W1, b1.reshape(1, 512))

  y2t = _wide_agg2(h1st.reshape(8 * NPAD, WF), combo3)
  zs = _tc_mm2(y2t.reshape(8, NPAD, WF), dinv, W2, b2.reshape(1, 512),
               W3.reshape(512, 1))

  zs16 = jnp.pad(zs, ((0, 0), (0, 15)))
  y3p = _scalar_agg(zs16, combo3s).reshape(2, NPAD, 16)[:, :, :1]

  batch_p = jnp.concatenate(
      [batch, jnp.full((NPAD - N,), 100, jnp.int32)]).reshape(1, NPAD)
  out = _tc_pool(y3p, zs, dinv, batch_p, b3.reshape(1, 1))
  return out[:, 0]
